# speedup vs baseline: 2.1060x; 2.1060x over previous
import functools

import jax
import jax.numpy as jnp
from jax import lax
from jax.experimental import pallas as pl
from jax.experimental.pallas import tpu as pltpu
from jax.experimental.pallas import tpu_sc as plsc

B, L, K, G, D, H, DH = 1, 2048, 16, 16, 768, 12, 64
INNER = H * DH
SCALE = DH ** -0.5
C = 1 + K + G
FF = 4 * D


_LB_A = 256


def _a_body(x_ref, g_ref, b_ref, wq_ref, wk_ref, wv_ref, gwk_ref, gwv_ref,
            q_ref, kv_ref, gkv_ref):
    x = x_ref[...]
    m = jnp.mean(x, -1, keepdims=True)
    v = jnp.mean((x - m) ** 2, -1, keepdims=True)
    sn = (x - m) * lax.rsqrt(v + 1e-5) * g_ref[...] + b_ref[...]
    f32 = jnp.float32
    q_ref[...] = jnp.dot(sn, wq_ref[...], preferred_element_type=f32)
    kv_ref[:, :INNER] = jnp.dot(sn, wk_ref[...], preferred_element_type=f32)
    kv_ref[:, INNER:] = jnp.dot(sn, wv_ref[...], preferred_element_type=f32)
    gkv_ref[:, :INNER] = jnp.dot(sn, gwk_ref[...], preferred_element_type=f32)
    gkv_ref[:, INNER:] = jnp.dot(sn, gwv_ref[...], preferred_element_type=f32)


def _proj_spatial(sp, p):
    grid = L // _LB_A
    row = lambda i: (i, 0)
    full = lambda i: (0, 0)
    return pl.pallas_call(
        _a_body,
        grid=grid,
        in_specs=[
            pl.BlockSpec((_LB_A, D), row),
            pl.BlockSpec((1, D), full),
            pl.BlockSpec((1, D), full),
            pl.BlockSpec((D, INNER), full),
            pl.BlockSpec((D, INNER), full),
            pl.BlockSpec((D, INNER), full),
            pl.BlockSpec((D, INNER), full),
            pl.BlockSpec((D, INNER), full),
        ],
        out_specs=[
            pl.BlockSpec((_LB_A, INNER), row),
            pl.BlockSpec((_LB_A, 2 * INNER), row),
            pl.BlockSpec((_LB_A, 2 * INNER), row),
        ],
        out_shape=[
            jax.ShapeDtypeStruct((L, INNER), jnp.float32),
            jax.ShapeDtypeStruct((L, 2 * INNER), jnp.float32),
            jax.ShapeDtypeStruct((L, 2 * INNER), jnp.float32),
        ],
    )(sp, p['ln_s_g'].reshape(1, D), p['ln_s_b'].reshape(1, D),
      p['sa_wq'], p['sa_wk'], p['sa_wv'], p['ga_wk'], p['ga_wv'])


def _g_body(x_ref, g_ref, b_ref, wk_ref, wv_ref, gwq_ref, gwk_ref, gwv_ref,
            qg_ref, kv_ref, gkv_ref):
    x = x_ref[...]
    m = jnp.mean(x, -1, keepdims=True)
    v = jnp.mean((x - m) ** 2, -1, keepdims=True)
    gn = (x - m) * lax.rsqrt(v + 1e-5) * g_ref[...] + b_ref[...]
    f32 = jnp.float32
    qg_ref[...] = jnp.dot(gn, gwq_ref[...], preferred_element_type=f32)
    kv_ref[:, :INNER] = jnp.dot(gn, wk_ref[...], preferred_element_type=f32)
    kv_ref[:, INNER:] = jnp.dot(gn, wv_ref[...], preferred_element_type=f32)
    gkv_ref[:, :INNER] = jnp.dot(gn, gwk_ref[...], preferred_element_type=f32)
    gkv_ref[:, INNER:] = jnp.dot(gn, gwv_ref[...], preferred_element_type=f32)


def _proj_global(gl, p):
    full = lambda: (0, 0)
    return pl.pallas_call(
        _g_body,
        in_specs=[
            pl.BlockSpec((G, D), full),
            pl.BlockSpec((1, D), full),
            pl.BlockSpec((1, D), full),
            pl.BlockSpec((D, INNER), full),
            pl.BlockSpec((D, INNER), full),
            pl.BlockSpec((D, INNER), full),
            pl.BlockSpec((D, INNER), full),
            pl.BlockSpec((D, INNER), full),
        ],
        out_specs=[
            pl.BlockSpec((G, INNER), full),
            pl.BlockSpec((G, 2 * INNER), full),
            pl.BlockSpec((G, 2 * INNER), full),
        ],
        out_shape=[
            jax.ShapeDtypeStruct((G, INNER), jnp.float32),
            jax.ShapeDtypeStruct((G, 2 * INNER), jnp.float32),
            jax.ShapeDtypeStruct((G, 2 * INNER), jnp.float32),
        ],
    )(gl, p['ln_g_g'].reshape(1, D), p['ln_g_b'].reshape(1, D),
      p['sa_wk'], p['sa_wv'], p['ga_wq'], p['ga_wk'], p['ga_wv'])


_NC, _NS = 2, 16
_NW = _NC * _NS
_ROWS = L * K
_RPW = _ROWS // _NW
_CHUNK = 64


def _sc_gather(kv, idx):
    mesh = plsc.VectorSubcoreMesh(core_axis_name="c", subcore_axis_name="s")

    @functools.partial(
        pl.kernel, mesh=mesh,
        out_type=jax.ShapeDtypeStruct((_ROWS, 2 * INNER), jnp.float32),
        scratch_types=[
            pltpu.VMEM((_CHUNK,), jnp.int32),
            pltpu.VMEM((_CHUNK, 2 * INNER), jnp.float32),
            pltpu.SemaphoreType.DMA,
        ],
    )
    def k(kv_hbm, idx_hbm, out_hbm, idx_v, rows_v, sem):
        wid = lax.axis_index("s") * _NC + lax.axis_index("c")
        base = wid * _RPW

        @pl.loop(0, _RPW // _CHUNK)
        def _(c):
            b = base + c * _CHUNK
            pltpu.sync_copy(idx_hbm.at[pl.ds(b, _CHUNK)], idx_v)
            pltpu.async_copy(kv_hbm.at[idx_v], rows_v, sem).wait()
            pltpu.sync_copy(rows_v, out_hbm.at[pl.ds(b, _CHUNK)])

    return k(kv, idx)


_LB_B = 128


def _head_sel(transpose):
    shape = (H, INNER) if transpose else (INNER, H)
    d_ax, h_ax = (1, 0) if transpose else (0, 1)
    d = lax.broadcasted_iota(jnp.int32, shape, d_ax)
    h = lax.broadcasted_iota(jnp.int32, shape, h_ax)
    return (d // DH == h).astype(jnp.float32)


def _b_body(q_ref, kv_ref, kvn_ref, kvg_ref, dist_ref, misc_ref, sp_ref,
            wo_ref, bo_ref, lg_ref, lb_ref, w1_ref, b1_ref, w2_ref, b2_ref,
            s_ref):
    f32 = jnp.float32
    q = q_ref[...]
    ks = [kv_ref[:, :INNER]]
    vs = [kv_ref[:, INNER:]]
    for k in range(K):
        ks.append(kvn_ref[k, :, :INNER])
        vs.append(kvn_ref[k, :, INNER:])
    for g in range(G):
        ks.append(jnp.broadcast_to(kvg_ref[g:g + 1, :INNER], (_LB_B, INNER)))
        vs.append(jnp.broadcast_to(kvg_ref[g:g + 1, INNER:], (_LB_B, INNER)))

    esel = _head_sel(False)
    p_stack = jnp.concatenate([q * kk for kk in ks], axis=0)
    logits = jnp.dot(p_stack, esel, preferred_element_type=f32) * SCALE

    inv2sig = 0.5 * jnp.exp(-2.0 * misc_ref[0:1, :H])
    gbias = misc_ref[1, 0]
    d = dist_ref[...]
    bias = [jnp.zeros((_LB_B, H), f32)]
    for k in range(K):
        bias.append(-(d[:, k:k + 1] ** 2) * inv2sig)
    gb = jnp.full((_LB_B, H), gbias, f32)
    for g in range(G):
        bias.append(gb)
    logits = logits + jnp.concatenate(bias, axis=0)

    m = logits[0:_LB_B]
    for c in range(1, C):
        m = jnp.maximum(m, logits[c * _LB_B:(c + 1) * _LB_B])
    e = jnp.exp(logits - jnp.concatenate([m] * C, axis=0))
    z = e[0:_LB_B]
    for c in range(1, C):
        z = z + e[c * _LB_B:(c + 1) * _LB_B]
    recip = 1.0 / z
    attn = e * jnp.concatenate([recip] * C, axis=0)

    a_exp = jnp.dot(attn, _head_sel(True), preferred_element_type=f32)
    prod = a_exp * jnp.concatenate(vs, axis=0)
    out = prod[0:_LB_B]
    for c in range(1, C):
        out = out + prod[c * _LB_B:(c + 1) * _LB_B]

    y = sp_ref[...] + jnp.dot(out, wo_ref[...], preferred_element_type=f32) \
        + bo_ref[...]
    mu = jnp.mean(y, -1, keepdims=True)
    var = jnp.mean((y - mu) ** 2, -1, keepdims=True)
    hn = (y - mu) * lax.rsqrt(var + 1e-5) * lg_ref[...] + lb_ref[...]
    t = jax.nn.gelu(jnp.dot(hn, w1_ref[...], preferred_element_type=f32)
                    + b1_ref[...])
    s_ref[...] = y + jnp.dot(t, w2_ref[...], preferred_element_type=f32) \
        + b2_ref[...]


def _local_block(q, kv, kvn3, kvg, dist, misc, sp, p):
    grid = L // _LB_B
    row = lambda i: (i, 0)
    full = lambda i: (0, 0)
    return pl.pallas_call(
        _b_body,
        grid=grid,
        in_specs=[
            pl.BlockSpec((_LB_B, INNER), row),
            pl.BlockSpec((_LB_B, 2 * INNER), row),
            pl.BlockSpec((K, _LB_B, 2 * INNER), lambda i: (0, i, 0)),
            pl.BlockSpec((G, 2 * INNER), full),
            pl.BlockSpec((_LB_B, K), row),
            pl.BlockSpec((8, 128), full),
            pl.BlockSpec((_LB_B, D), row),
            pl.BlockSpec((INNER, D), full),
            pl.BlockSpec((1, D), full),
            pl.BlockSpec((1, D), full),
            pl.BlockSpec((1, D), full),
            pl.BlockSpec((D, FF), full),
            pl.BlockSpec((1, FF), full),
            pl.BlockSpec((FF, D), full),
            pl.BlockSpec((1, D), full),
        ],
        out_specs=pl.BlockSpec((_LB_B, D), row),
        out_shape=jax.ShapeDtypeStruct((L, D), jnp.float32),
    )(q, kv, kvn3, kvg, dist, misc, sp,
      p['sa_wo'], p['sa_bo'].reshape(1, D),
      p['ffs_ln_g'].reshape(1, D), p['ffs_ln_b'].reshape(1, D),
      p['ffs_w1'], p['ffs_b1'].reshape(1, FF),
      p['ffs_w2'], p['ffs_b2'].reshape(1, D))


def _c_body(qg_ref, gkv_ref, gkvg_ref, misc_ref, gl_ref,
            wo_ref, bo_ref, lng_ref, lnb_ref, w1_ref, b1_ref, w2_ref, b2_ref,
            g_out_ref):
    f32 = jnp.float32
    gbias = misc_ref[2, 0]
    qg = qg_ref[...]
    k_all = jnp.concatenate([gkv_ref[:, :INNER], gkvg_ref[:, :INNER]], axis=0)
    v_all = jnp.concatenate([gkv_ref[:, INNER:], gkvg_ref[:, INNER:]], axis=0)
    outs = []
    for h in range(H):
        sl = slice(h * DH, (h + 1) * DH)
        lg = lax.dot_general(qg[:, sl], k_all[:, sl],
                             (((1,), (1,)), ((), ())),
                             preferred_element_type=f32) * SCALE + gbias
        lg = lg - jnp.max(lg, axis=-1, keepdims=True)
        e = jnp.exp(lg)
        a = e / jnp.sum(e, axis=-1, keepdims=True)
        outs.append(jnp.dot(a, v_all[:, sl], preferred_element_type=f32))
    og = jnp.concatenate(outs, axis=1)
    y = gl_ref[...] + jnp.dot(og, wo_ref[...], preferred_element_type=f32) \
        + bo_ref[...]
    mu = jnp.mean(y, -1, keepdims=True)
    var = jnp.mean((y - mu) ** 2, -1, keepdims=True)
    hn = (y - mu) * lax.rsqrt(var + 1e-5) * lng_ref[...] + lnb_ref[...]
    t = jax.nn.gelu(jnp.dot(hn, w1_ref[...], preferred_element_type=f32)
                    + b1_ref[...])
    g_out_ref[...] = y + jnp.dot(t, w2_ref[...], preferred_element_type=f32) \
        + b2_ref[...]


def _global_block(qg, gkv, gkvg, misc, gl, p):
    full2 = lambda: (0, 0)
    return pl.pallas_call(
        _c_body,
        in_specs=[
            pl.BlockSpec((G, INNER), full2),
            pl.BlockSpec((L, 2 * INNER), full2),
            pl.BlockSpec((G, 2 * INNER), full2),
            pl.BlockSpec((8, 128), full2),
            pl.BlockSpec((G, D), full2),
            pl.BlockSpec((INNER, D), full2),
            pl.BlockSpec((1, D), full2),
            pl.BlockSpec((1, D), full2),
            pl.BlockSpec((1, D), full2),
            pl.BlockSpec((D, FF), full2),
            pl.BlockSpec((1, FF), full2),
            pl.BlockSpec((FF, D), full2),
            pl.BlockSpec((1, D), full2),
        ],
        out_specs=pl.BlockSpec((G, D), full2),
        out_shape=jax.ShapeDtypeStruct((G, D), jnp.float32),
    )(qg, gkv, gkvg, misc, gl,
      p['ga_wo'], p['ga_bo'].reshape(1, D),
      p['ffg_ln_g'].reshape(1, D), p['ffg_ln_b'].reshape(1, D),
      p['ffg_w1'], p['ffg_b1'].reshape(1, FF),
      p['ffg_w2'], p['ffg_b2'].reshape(1, D))


def kernel(spatial, topk_indices, rpe, self_rpe, distances, global_latents,
           params):
    p = params
    sp = spatial.reshape(L, D)
    gl = global_latents.reshape(G, D)
    dist = distances.reshape(L, K)
    idx_t = topk_indices.reshape(L, K).astype(jnp.int32).T.reshape(-1)
    misc = jnp.zeros((8, 128), jnp.float32)
    misc = misc.at[0, :H].set(p['sa_log_sigma'])
    misc = misc.at[1, 0].set(p['sa_gbias'])
    misc = misc.at[2, 0].set(p['ga_gbias'])

    q, kv, gkv = _proj_spatial(sp, p)
    qg, kvg, gkvg = _proj_global(gl, p)
    kvn3 = _sc_gather(kv, idx_t).reshape(K, L, 2 * INNER)
    s = _local_block(q, kv, kvn3, kvg, dist, misc, sp, p)
    g = _global_block(qg, gkv, gkvg, misc, gl, p)
    return s.reshape(B, L, D), g.reshape(B, G, D)


# speedup vs baseline: 2.5848x; 1.2274x over previous
import functools

import jax
import jax.numpy as jnp
from jax import lax
from jax.experimental import pallas as pl
from jax.experimental.pallas import tpu as pltpu
from jax.experimental.pallas import tpu_sc as plsc

B, L, K, G, D, H, DH = 1, 2048, 16, 16, 768, 12, 64
INNER = H * DH
SCALE = DH ** -0.5
C = 1 + K + G
FF = 4 * D


_LB_A = 256


def _a_body(x_ref, g_ref, b_ref, wq_ref, wk_ref, wv_ref, gwk_ref, gwv_ref,
            q_ref, kv_ref, gkv_ref):
    x = x_ref[...]
    m = jnp.mean(x, -1, keepdims=True)
    v = jnp.mean((x - m) ** 2, -1, keepdims=True)
    sn = (x - m) * lax.rsqrt(v + 1e-5) * g_ref[...] + b_ref[...]
    f32 = jnp.float32
    q_ref[...] = jnp.dot(sn, wq_ref[...], preferred_element_type=f32)
    ku = lax.bitcast_convert_type(
        jnp.dot(sn, wk_ref[...], preferred_element_type=f32), jnp.uint32)
    vu = lax.bitcast_convert_type(
        jnp.dot(sn, wv_ref[...], preferred_element_type=f32), jnp.uint32)
    w = ((ku + 0x8000) & jnp.uint32(0xffff0000)) | ((vu + 0x8000) >> 16)
    kv_ref[...] = lax.bitcast_convert_type(w, jnp.int32)
    gkv_ref[:, :INNER] = jnp.dot(sn, gwk_ref[...], preferred_element_type=f32)
    gkv_ref[:, INNER:] = jnp.dot(sn, gwv_ref[...], preferred_element_type=f32)


def _proj_spatial(sp, p):
    grid = L // _LB_A
    row = lambda i: (i, 0)
    full = lambda i: (0, 0)
    return pl.pallas_call(
        _a_body,
        grid=grid,
        in_specs=[
            pl.BlockSpec((_LB_A, D), row),
            pl.BlockSpec((1, D), full),
            pl.BlockSpec((1, D), full),
            pl.BlockSpec((D, INNER), full),
            pl.BlockSpec((D, INNER), full),
            pl.BlockSpec((D, INNER), full),
            pl.BlockSpec((D, INNER), full),
            pl.BlockSpec((D, INNER), full),
        ],
        out_specs=[
            pl.BlockSpec((_LB_A, INNER), row),
            pl.BlockSpec((_LB_A, INNER), row),
            pl.BlockSpec((_LB_A, 2 * INNER), row),
        ],
        out_shape=[
            jax.ShapeDtypeStruct((L, INNER), jnp.float32),
            jax.ShapeDtypeStruct((L, INNER), jnp.int32),
            jax.ShapeDtypeStruct((L, 2 * INNER), jnp.float32),
        ],
    )(sp, p['ln_s_g'].reshape(1, D), p['ln_s_b'].reshape(1, D),
      p['sa_wq'], p['sa_wk'], p['sa_wv'], p['ga_wk'], p['ga_wv'])


def _g_body(x_ref, g_ref, b_ref, wk_ref, wv_ref, gwq_ref, gwk_ref, gwv_ref,
            qg_ref, kv_ref, gkv_ref):
    x = x_ref[...]
    m = jnp.mean(x, -1, keepdims=True)
    v = jnp.mean((x - m) ** 2, -1, keepdims=True)
    gn = (x - m) * lax.rsqrt(v + 1e-5) * g_ref[...] + b_ref[...]
    f32 = jnp.float32
    qg_ref[...] = jnp.dot(gn, gwq_ref[...], preferred_element_type=f32)
    kv_ref[:, :INNER] = jnp.dot(gn, wk_ref[...], preferred_element_type=f32)
    kv_ref[:, INNER:] = jnp.dot(gn, wv_ref[...], preferred_element_type=f32)
    gkv_ref[:, :INNER] = jnp.dot(gn, gwk_ref[...], preferred_element_type=f32)
    gkv_ref[:, INNER:] = jnp.dot(gn, gwv_ref[...], preferred_element_type=f32)


def _proj_global(gl, p):
    full = lambda: (0, 0)
    return pl.pallas_call(
        _g_body,
        in_specs=[
            pl.BlockSpec((G, D), full),
            pl.BlockSpec((1, D), full),
            pl.BlockSpec((1, D), full),
            pl.BlockSpec((D, INNER), full),
            pl.BlockSpec((D, INNER), full),
            pl.BlockSpec((D, INNER), full),
            pl.BlockSpec((D, INNER), full),
            pl.BlockSpec((D, INNER), full),
        ],
        out_specs=[
            pl.BlockSpec((G, INNER), full),
            pl.BlockSpec((G, 2 * INNER), full),
            pl.BlockSpec((G, 2 * INNER), full),
        ],
        out_shape=[
            jax.ShapeDtypeStruct((G, INNER), jnp.float32),
            jax.ShapeDtypeStruct((G, 2 * INNER), jnp.float32),
            jax.ShapeDtypeStruct((G, 2 * INNER), jnp.float32),
        ],
    )(gl, p['ln_g_g'].reshape(1, D), p['ln_g_b'].reshape(1, D),
      p['sa_wk'], p['sa_wv'], p['ga_wq'], p['ga_wk'], p['ga_wv'])


_NC, _NS = 2, 16
_NW = _NC * _NS
_ROWS = L * K
_RPW = _ROWS // _NW
_CHUNK = 64


def _sc_gather(kv, idx):
    mesh = plsc.VectorSubcoreMesh(core_axis_name="c", subcore_axis_name="s")

    @functools.partial(
        pl.kernel, mesh=mesh,
        out_type=jax.ShapeDtypeStruct((_ROWS, INNER), jnp.int32),
        scratch_types=[
            pltpu.VMEM((_CHUNK,), jnp.int32),
            pltpu.VMEM((_CHUNK, INNER), jnp.int32),
            pltpu.SemaphoreType.DMA,
        ],
    )
    def k(kv_hbm, idx_hbm, out_hbm, idx_v, rows_v, sem):
        wid = lax.axis_index("s") * _NC + lax.axis_index("c")
        base = wid * _RPW

        @pl.loop(0, _RPW // _CHUNK)
        def _(c):
            b = base + c * _CHUNK
            pltpu.sync_copy(idx_hbm.at[pl.ds(b, _CHUNK)], idx_v)
            pltpu.async_copy(kv_hbm.at[idx_v], rows_v, sem).wait()
            pltpu.sync_copy(rows_v, out_hbm.at[pl.ds(b, _CHUNK)])

    return k(kv, idx)


_LB_B = 128


def _head_sel(transpose):
    shape = (H, INNER) if transpose else (INNER, H)
    d_ax, h_ax = (1, 0) if transpose else (0, 1)
    d = lax.broadcasted_iota(jnp.int32, shape, d_ax)
    h = lax.broadcasted_iota(jnp.int32, shape, h_ax)
    return (d // DH == h).astype(jnp.float32)


def _b_body(q_ref, kv_ref, kvn_ref, kvg_ref, dist_ref, misc_ref, sp_ref,
            wo_ref, bo_ref, lg_ref, lb_ref, w1_ref, b1_ref, w2_ref, b2_ref,
            s_ref):
    f32 = jnp.float32

    def unpack(w):
        wu = lax.bitcast_convert_type(w, jnp.uint32)
        kk = lax.bitcast_convert_type(wu & jnp.uint32(0xffff0000), f32)
        vv = lax.bitcast_convert_type(wu << 16, f32)
        return kk, vv

    q = q_ref[...]
    ks, vs = [], []
    k0, v0 = unpack(kv_ref[...])
    ks.append(k0)
    vs.append(v0)
    for k in range(K):
        kk, vv = unpack(kvn_ref[k])
        ks.append(kk)
        vs.append(vv)
    for g in range(G):
        ks.append(jnp.broadcast_to(kvg_ref[g:g + 1, :INNER], (_LB_B, INNER)))
        vs.append(jnp.broadcast_to(kvg_ref[g:g + 1, INNER:], (_LB_B, INNER)))

    esel = _head_sel(False)
    p_stack = jnp.concatenate([q * kk for kk in ks], axis=0)
    logits = jnp.dot(p_stack, esel, preferred_element_type=f32) * SCALE

    inv2sig = 0.5 * jnp.exp(-2.0 * misc_ref[0:1, :H])
    gbias = misc_ref[1, 0]
    d = dist_ref[...]
    bias = [jnp.zeros((_LB_B, H), f32)]
    for k in range(K):
        bias.append(-(d[:, k:k + 1] ** 2) * inv2sig)
    gb = jnp.full((_LB_B, H), gbias, f32)
    for g in range(G):
        bias.append(gb)
    logits = logits + jnp.concatenate(bias, axis=0)

    m = logits[0:_LB_B]
    for c in range(1, C):
        m = jnp.maximum(m, logits[c * _LB_B:(c + 1) * _LB_B])
    e = jnp.exp(logits - jnp.concatenate([m] * C, axis=0))
    z = e[0:_LB_B]
    for c in range(1, C):
        z = z + e[c * _LB_B:(c + 1) * _LB_B]
    recip = 1.0 / z
    attn = e * jnp.concatenate([recip] * C, axis=0)

    a_exp = jnp.dot(attn, _head_sel(True), preferred_element_type=f32)
    prod = a_exp * jnp.concatenate(vs, axis=0)
    out = prod[0:_LB_B]
    for c in range(1, C):
        out = out + prod[c * _LB_B:(c + 1) * _LB_B]

    y = sp_ref[...] + jnp.dot(out, wo_ref[...], preferred_element_type=f32) \
        + bo_ref[...]
    mu = jnp.mean(y, -1, keepdims=True)
    var = jnp.mean((y - mu) ** 2, -1, keepdims=True)
    hn = (y - mu) * lax.rsqrt(var + 1e-5) * lg_ref[...] + lb_ref[...]
    t = jax.nn.gelu(jnp.dot(hn, w1_ref[...], preferred_element_type=f32)
                    + b1_ref[...])
    s_ref[...] = y + jnp.dot(t, w2_ref[...], preferred_element_type=f32) \
        + b2_ref[...]


def _local_block(q, kv, kvn3, kvg, dist, misc, sp, p):
    grid = L // _LB_B
    row = lambda i: (i, 0)
    full = lambda i: (0, 0)
    return pl.pallas_call(
        _b_body,
        grid=grid,
        in_specs=[
            pl.BlockSpec((_LB_B, INNER), row),
            pl.BlockSpec((_LB_B, INNER), row),
            pl.BlockSpec((K, _LB_B, INNER), lambda i: (0, i, 0)),
            pl.BlockSpec((G, 2 * INNER), full),
            pl.BlockSpec((_LB_B, K), row),
            pl.BlockSpec((8, 128), full),
            pl.BlockSpec((_LB_B, D), row),
            pl.BlockSpec((INNER, D), full),
            pl.BlockSpec((1, D), full),
            pl.BlockSpec((1, D), full),
            pl.BlockSpec((1, D), full),
            pl.BlockSpec((D, FF), full),
            pl.BlockSpec((1, FF), full),
            pl.BlockSpec((FF, D), full),
            pl.BlockSpec((1, D), full),
        ],
        out_specs=pl.BlockSpec((_LB_B, D), row),
        out_shape=jax.ShapeDtypeStruct((L, D), jnp.float32),
    )(q, kv, kvn3, kvg, dist, misc, sp,
      p['sa_wo'], p['sa_bo'].reshape(1, D),
      p['ffs_ln_g'].reshape(1, D), p['ffs_ln_b'].reshape(1, D),
      p['ffs_w1'], p['ffs_b1'].reshape(1, FF),
      p['ffs_w2'], p['ffs_b2'].reshape(1, D))


def _c_body(qg_ref, gkv_ref, gkvg_ref, misc_ref, gl_ref,
            wo_ref, bo_ref, lng_ref, lnb_ref, w1_ref, b1_ref, w2_ref, b2_ref,
            g_out_ref):
    f32 = jnp.float32
    gbias = misc_ref[2, 0]
    qg = qg_ref[...]
    k_all = jnp.concatenate([gkv_ref[:, :INNER], gkvg_ref[:, :INNER]], axis=0)
    v_all = jnp.concatenate([gkv_ref[:, INNER:], gkvg_ref[:, INNER:]], axis=0)
    outs = []
    for h in range(H):
        sl = slice(h * DH, (h + 1) * DH)
        lg = lax.dot_general(qg[:, sl], k_all[:, sl],
                             (((1,), (1,)), ((), ())),
                             preferred_element_type=f32) * SCALE + gbias
        lg = lg - jnp.max(lg, axis=-1, keepdims=True)
        e = jnp.exp(lg)
        a = e / jnp.sum(e, axis=-1, keepdims=True)
        outs.append(jnp.dot(a, v_all[:, sl], preferred_element_type=f32))
    og = jnp.concatenate(outs, axis=1)
    y = gl_ref[...] + jnp.dot(og, wo_ref[...], preferred_element_type=f32) \
        + bo_ref[...]
    mu = jnp.mean(y, -1, keepdims=True)
    var = jnp.mean((y - mu) ** 2, -1, keepdims=True)
    hn = (y - mu) * lax.rsqrt(var + 1e-5) * lng_ref[...] + lnb_ref[...]
    t = jax.nn.gelu(jnp.dot(hn, w1_ref[...], preferred_element_type=f32)
                    + b1_ref[...])
    g_out_ref[...] = y + jnp.dot(t, w2_ref[...], preferred_element_type=f32) \
        + b2_ref[...]


def _global_block(qg, gkv, gkvg, misc, gl, p):
    full2 = lambda: (0, 0)
    return pl.pallas_call(
        _c_body,
        in_specs=[
            pl.BlockSpec((G, INNER), full2),
            pl.BlockSpec((L, 2 * INNER), full2),
            pl.BlockSpec((G, 2 * INNER), full2),
            pl.BlockSpec((8, 128), full2),
            pl.BlockSpec((G, D), full2),
            pl.BlockSpec((INNER, D), full2),
            pl.BlockSpec((1, D), full2),
            pl.BlockSpec((1, D), full2),
            pl.BlockSpec((1, D), full2),
            pl.BlockSpec((D, FF), full2),
            pl.BlockSpec((1, FF), full2),
            pl.BlockSpec((FF, D), full2),
            pl.BlockSpec((1, D), full2),
        ],
        out_specs=pl.BlockSpec((G, D), full2),
        out_shape=jax.ShapeDtypeStruct((G, D), jnp.float32),
    )(qg, gkv, gkvg, misc, gl,
      p['ga_wo'], p['ga_bo'].reshape(1, D),
      p['ffg_ln_g'].reshape(1, D), p['ffg_ln_b'].reshape(1, D),
      p['ffg_w1'], p['ffg_b1'].reshape(1, FF),
      p['ffg_w2'], p['ffg_b2'].reshape(1, D))


def kernel(spatial, topk_indices, rpe, self_rpe, distances, global_latents,
           params):
    p = params
    sp = spatial.reshape(L, D)
    gl = global_latents.reshape(G, D)
    dist = distances.reshape(L, K)
    idx_t = topk_indices.reshape(L, K).astype(jnp.int32).T.reshape(-1)
    misc = jnp.zeros((8, 128), jnp.float32)
    misc = misc.at[0, :H].set(p['sa_log_sigma'])
    misc = misc.at[1, 0].set(p['sa_gbias'])
    misc = misc.at[2, 0].set(p['ga_gbias'])

    q, kv, gkv = _proj_spatial(sp, p)
    qg, kvg, gkvg = _proj_global(gl, p)
    kvn3 = _sc_gather(kv, idx_t).reshape(K, L, INNER)
    s = _local_block(q, kv, kvn3, kvg, dist, misc, sp, p)
    g = _global_block(qg, gkv, gkvg, misc, gl, p)
    return s.reshape(B, L, D), g.reshape(B, G, D)
